# baseline (speedup 1.0000x reference)
import functools

import jax
import jax.numpy as jnp
from jax import lax
from jax.experimental import pallas as pl
from jax.experimental.pallas import tpu as pltpu
from jax.experimental.pallas import tpu_sc as plsc

U = 50000
E = 3200000
F = 16
G = 64
DEP = 2
CC = 0.85
EPS = 1e-16

NC = 2
NS = 16
NW = NC * NS
L = 16

EPT = E // NW
CE = 2000
EPT2 = E // 2
CE2 = 4000

UB = 2000
NU = U // UB


def _sc_params():
    return pltpu.CompilerParams(needs_layout_passes=False,
                                use_tc_tiling_on_sc=False)


def _mesh():
    return plsc.VectorSubcoreMesh(core_axis_name="c", subcore_axis_name="s")


def _wid():
    return lax.axis_index("s") * NC + lax.axis_index("c")


def _sc_edge_sums(r1, w1, r2, w2):
    @functools.partial(
        pl.kernel,
        mesh=_mesh(),
        out_type=(
            jax.ShapeDtypeStruct((NW, U), jnp.float32),
            jax.ShapeDtypeStruct((NW, U), jnp.float32),
        ),
        scratch_types=[
            pltpu.VMEM((U,), jnp.float32),
            pltpu.VMEM((U,), jnp.float32),
            pltpu.VMEM((2, CE), jnp.int32),
            pltpu.VMEM((2, CE), jnp.float32),
            pltpu.VMEM((2, CE), jnp.int32),
            pltpu.VMEM((2, CE), jnp.float32),
            pltpu.SemaphoreType.DMA,
            pltpu.SemaphoreType.DMA,
        ],
        compiler_params=_sc_params(),
    )
    def k(r1_h, w1_h, r2_h, w2_h, p1_h, p2_h, acc1, acc2, i1v, v1v, i2v, v2v,
          sem0, sem1):
        wid = _wid()
        base = wid * EPT
        sems = (sem0, sem1)

        def start(b, chunk):
            off = base + chunk * CE
            pltpu.async_copy(r1_h.at[pl.ds(off, CE)], i1v.at[b], sems[b])
            pltpu.async_copy(w1_h.at[pl.ds(off, CE)], v1v.at[b], sems[b])
            pltpu.async_copy(r2_h.at[pl.ds(off, CE)], i2v.at[b], sems[b])
            pltpu.async_copy(w2_h.at[pl.ds(off, CE)], v2v.at[b], sems[b])

        def drain(b):
            pltpu.make_async_copy(r1_h.at[pl.ds(0, CE)], i1v.at[b], sems[b]).wait()
            pltpu.make_async_copy(w1_h.at[pl.ds(0, CE)], v1v.at[b], sems[b]).wait()
            pltpu.make_async_copy(r2_h.at[pl.ds(0, CE)], i2v.at[b], sems[b]).wait()
            pltpu.make_async_copy(w2_h.at[pl.ds(0, CE)], v2v.at[b], sems[b]).wait()

        start(0, 0)
        start(1, 1)

        @plsc.parallel_loop(0, U // L, unroll=8)
        def _(i):
            z = jnp.zeros((L,), jnp.float32)
            acc1[pl.ds(i * L, L)] = z
            acc2[pl.ds(i * L, L)] = z

        @pl.loop(0, EPT // CE, step=2)
        def _(c):
            for b in range(2):
                drain(b)

                @plsc.parallel_loop(0, CE // L, unroll=8)
                def _(i):
                    sl = pl.ds(i * L, L)
                    plsc.addupdate_scatter(acc1, [i1v[b, sl]],
                                           jnp.exp(v1v[b, sl]))
                    plsc.addupdate_scatter(acc2, [i2v[b, sl]],
                                           jnp.exp(v2v[b, sl]))

                nxt = c + b + 2

                @pl.when(nxt < EPT // CE)
                def _():
                    start(b, nxt)

        pltpu.sync_copy(acc1, p1_h.at[wid])
        pltpu.sync_copy(acc2, p2_h.at[wid])

    return k(r1, w1, r2, w2)


def _sc_lfw1(inv1, r1, w1):
    @functools.partial(
        pl.kernel,
        mesh=_mesh(),
        out_type=jax.ShapeDtypeStruct((E,), jnp.float32),
        scratch_types=[
            pltpu.VMEM((U,), jnp.float32),
            pltpu.VMEM((2, CE), jnp.int32),
            pltpu.VMEM((2, CE), jnp.float32),
            pltpu.VMEM((2, CE), jnp.float32),
            pltpu.SemaphoreType.DMA,
            pltpu.SemaphoreType.DMA,
            pltpu.SemaphoreType.DMA,
            pltpu.SemaphoreType.DMA,
        ],
        compiler_params=_sc_params(),
    )
    def k(inv_h, r_h, w_h, out_h, sv, iv, wv, ov, sem0, sem1, osem0, osem1):
        wid = _wid()
        base = wid * EPT
        sems = (sem0, sem1)
        osems = (osem0, osem1)
        pltpu.sync_copy(inv_h, sv)

        def start(b, chunk):
            off = base + chunk * CE
            pltpu.async_copy(r_h.at[pl.ds(off, CE)], iv.at[b], sems[b])
            pltpu.async_copy(w_h.at[pl.ds(off, CE)], wv.at[b], sems[b])

        def drain(b):
            pltpu.make_async_copy(r_h.at[pl.ds(0, CE)], iv.at[b], sems[b]).wait()
            pltpu.make_async_copy(w_h.at[pl.ds(0, CE)], wv.at[b], sems[b]).wait()

        start(0, 0)
        start(1, 1)

        @pl.loop(0, EPT // CE, step=2)
        def _(c):
            for b in range(2):
                drain(b)
                chunk = c + b

                @pl.when(chunk >= 2)
                def _():
                    pltpu.make_async_copy(ov.at[b], out_h.at[pl.ds(0, CE)],
                                          osems[b]).wait()

                @plsc.parallel_loop(0, CE // L, unroll=8)
                def _(i):
                    sl = pl.ds(i * L, L)
                    d = plsc.load_gather(sv, [iv[b, sl]])
                    ov[b, sl] = jnp.exp(wv[b, sl]) * d

                off = base + chunk * CE
                pltpu.async_copy(ov.at[b], out_h.at[pl.ds(off, CE)], osems[b])
                nxt = chunk + 2

                @pl.when(nxt < EPT // CE)
                def _():
                    start(b, nxt)

        for b in range(2):
            pltpu.make_async_copy(ov.at[b], out_h.at[pl.ds(0, CE)],
                                  osems[b]).wait()

    return k(inv1, r1, w1)


def _sc_lfw2_fa0(inv2, r2, w2, r1, lfw1):
    @functools.partial(
        pl.kernel,
        mesh=_mesh(),
        out_type=(
            jax.ShapeDtypeStruct((E,), jnp.float32),
            jax.ShapeDtypeStruct((NW, U), jnp.float32),
        ),
        scratch_types=[
            pltpu.VMEM((U,), jnp.float32),
            pltpu.VMEM((U,), jnp.float32),
            pltpu.VMEM((2, CE), jnp.int32),
            pltpu.VMEM((2, CE), jnp.float32),
            pltpu.VMEM((2, CE), jnp.int32),
            pltpu.VMEM((2, CE), jnp.float32),
            pltpu.VMEM((2, CE), jnp.float32),
            pltpu.SemaphoreType.DMA,
            pltpu.SemaphoreType.DMA,
            pltpu.SemaphoreType.DMA,
            pltpu.SemaphoreType.DMA,
        ],
        compiler_params=_sc_params(),
    )
    def k(inv_h, r2_h, w2_h, r1_h, lfw1_h, out_h, fp_h,
          sv, facc, i2v, w2v, i1v, l1v, ov, sem0, sem1, osem0, osem1):
        wid = _wid()
        base = wid * EPT
        sems = (sem0, sem1)
        osems = (osem0, osem1)
        pltpu.sync_copy(inv_h, sv)

        def start(b, chunk):
            off = base + chunk * CE
            pltpu.async_copy(r2_h.at[pl.ds(off, CE)], i2v.at[b], sems[b])
            pltpu.async_copy(w2_h.at[pl.ds(off, CE)], w2v.at[b], sems[b])
            pltpu.async_copy(r1_h.at[pl.ds(off, CE)], i1v.at[b], sems[b])
            pltpu.async_copy(lfw1_h.at[pl.ds(off, CE)], l1v.at[b], sems[b])

        def drain(b):
            pltpu.make_async_copy(r2_h.at[pl.ds(0, CE)], i2v.at[b], sems[b]).wait()
            pltpu.make_async_copy(w2_h.at[pl.ds(0, CE)], w2v.at[b], sems[b]).wait()
            pltpu.make_async_copy(r1_h.at[pl.ds(0, CE)], i1v.at[b], sems[b]).wait()
            pltpu.make_async_copy(lfw1_h.at[pl.ds(0, CE)], l1v.at[b], sems[b]).wait()

        start(0, 0)
        start(1, 1)

        @plsc.parallel_loop(0, U // L, unroll=8)
        def _(i):
            facc[pl.ds(i * L, L)] = jnp.zeros((L,), jnp.float32)

        @pl.loop(0, EPT // CE, step=2)
        def _(c):
            for b in range(2):
                drain(b)
                chunk = c + b

                @pl.when(chunk >= 2)
                def _():
                    pltpu.make_async_copy(ov.at[b], out_h.at[pl.ds(0, CE)],
                                          osems[b]).wait()

                @plsc.parallel_loop(0, CE // L, unroll=8)
                def _(i):
                    sl = pl.ds(i * L, L)
                    d = plsc.load_gather(sv, [i2v[b, sl]])
                    o = jnp.exp(w2v[b, sl]) * d
                    ov[b, sl] = o
                    plsc.addupdate_scatter(facc, [i1v[b, sl]], o * l1v[b, sl])

                off = base + chunk * CE
                pltpu.async_copy(ov.at[b], out_h.at[pl.ds(off, CE)], osems[b])
                nxt = chunk + 2

                @pl.when(nxt < EPT // CE)
                def _():
                    start(b, nxt)

        for b in range(2):
            pltpu.make_async_copy(ov.at[b], out_h.at[pl.ds(0, CE)],
                                  osems[b]).wait()
        pltpu.sync_copy(facc, fp_h.at[wid])

    return k(inv2, r2, w2, r1, lfw1)


def _sc_spmm(srcT, packed, wgt, gather_hi, paired_src=False):
    NCH = EPT2 // CE2

    scratch = [
        pltpu.VMEM((U,), jnp.float32),
        pltpu.VMEM((U,), jnp.float32),
        pltpu.VMEM((CE2,), jnp.int32),
        pltpu.VMEM((CE2,), jnp.float32),
        pltpu.VMEM((CE2,), jnp.int32),
        pltpu.VMEM((CE2,), jnp.float32),
        pltpu.SemaphoreType.DMA,
        pltpu.SemaphoreType.DMA,
    ]
    if paired_src:
        scratch.insert(2, pltpu.VMEM((UB,), jnp.float32))

    @functools.partial(
        pl.kernel,
        mesh=_mesh(),
        out_type=jax.ShapeDtypeStruct((2, F, U), jnp.float32),
        scratch_types=scratch,
        compiler_params=_sc_params(),
    )
    def k(srcT_h, pk_h, w_h, out_h, col, acc, *rest):
        if paired_src:
            tmp, pk0, w0, pk1, w1, sem0, sem1 = rest
        else:
            pk0, w0, pk1, w1, sem0, sem1 = rest
        wid = _wid()
        d = wid % F
        g = wid // F
        base = g * EPT2
        bufs = ((pk0, w0, sem0), (pk1, w1, sem1))

        def start(b, chunk):
            pkb, wb, semb = bufs[b]
            off = base + chunk * CE2
            pltpu.async_copy(pk_h.at[pl.ds(off, CE2)], pkb, semb)
            pltpu.async_copy(w_h.at[pl.ds(off, CE2)], wb, semb)

        def drain(b):
            pkb, wb, semb = bufs[b]
            pltpu.make_async_copy(pk_h.at[pl.ds(0, CE2)], pkb, semb).wait()
            pltpu.make_async_copy(w_h.at[pl.ds(0, CE2)], wb, semb).wait()

        start(0, 0)
        start(1, 1)

        if paired_src:
            @pl.loop(0, NU)
            def _(j):
                pltpu.sync_copy(srcT_h.at[0, d, pl.ds(j * UB, UB)],
                                col.at[pl.ds(j * UB, UB)])
                pltpu.sync_copy(srcT_h.at[1, d, pl.ds(j * UB, UB)], tmp)

                @plsc.parallel_loop(0, UB // L, unroll=8)
                def _(i):
                    sl = pl.ds(j * UB + i * L, L)
                    col[sl] = col[sl] + tmp[pl.ds(i * L, L)]
        else:
            @pl.loop(0, NU)
            def _(j):
                pltpu.sync_copy(srcT_h.at[j, d], col.at[pl.ds(j * UB, UB)])

        @plsc.parallel_loop(0, U // L, unroll=8)
        def _(i):
            acc[pl.ds(i * L, L)] = jnp.zeros((L,), jnp.float32)

        @pl.loop(0, NCH, step=2)
        def _(c):
            for b in range(2):
                pkb, wb, _ = bufs[b]
                drain(b)

                @plsc.parallel_loop(0, CE2 // L, unroll=32)
                def _(i):
                    sl = pl.ds(i * L, L)
                    pk = pkb[sl]
                    hi = lax.shift_right_logical(pk, 16)
                    lo = lax.bitwise_and(pk, 0xFFFF)
                    if gather_hi:
                        v = plsc.load_gather(col, [lo])
                        plsc.addupdate_scatter(acc, [hi], v * wb[sl])
                    else:
                        v = plsc.load_gather(col, [hi])
                        plsc.addupdate_scatter(acc, [lo], v * wb[sl])

                nxt = c + b + 2

                @pl.when(nxt < NCH)
                def _():
                    start(b, nxt)

        pltpu.sync_copy(acc, out_h.at[g, d])

    return k(srcT, packed, wgt)


def _tc_pack(r1, r2):
    r1m = r1.reshape(U, G)
    r2m = r2.reshape(U, G)

    def body(a_ref, b_ref, o_ref):
        o_ref[...] = lax.bitwise_or(lax.shift_left(a_ref[...], 16), b_ref[...])

    out = pl.pallas_call(
        body,
        grid=(NU,),
        in_specs=[
            pl.BlockSpec((UB, G), lambda i: (i, 0)),
            pl.BlockSpec((UB, G), lambda i: (i, 0)),
        ],
        out_specs=pl.BlockSpec((UB, G), lambda i: (i, 0)),
        out_shape=jax.ShapeDtypeStruct((U, G), jnp.int32),
    )(r1m, r2m)
    return out.reshape(E)


def _tc_prep_sums(la1, la2, p1, p2):
    def body(la1_ref, la2_ref, p1_ref, p2_ref, inv1_ref, inv2_ref, cs2_ref):
        i = pl.program_id(0)
        e1 = jnp.exp(la1_ref[...])
        rs = jnp.sum(e1, axis=1, keepdims=True)
        p1b = p1_ref[...].reshape(NW, UB)
        p2b = p2_ref[...].reshape(NW, UB)
        ps1 = jnp.transpose(jnp.sum(p1b, axis=0, keepdims=True), (1, 0))
        ps2 = jnp.transpose(jnp.sum(p2b, axis=0, keepdims=True), (1, 0))
        inv1_ref[...] = 1.0 / (rs + EPS + ps1)
        inv2_ref[...] = 1.0 / (EPS + ps2)
        part = jnp.sum(jnp.exp(la2_ref[...]), axis=0, keepdims=True)

        @pl.when(i == 0)
        def _():
            cs2_ref[...] = part

        @pl.when(i != 0)
        def _():
            cs2_ref[...] += part

    p1 = p1.reshape(NW, NU, 1, UB)
    p2 = p2.reshape(NW, NU, 1, UB)
    return pl.pallas_call(
        body,
        grid=(NU,),
        in_specs=[
            pl.BlockSpec((UB, G), lambda i: (i, 0)),
            pl.BlockSpec((UB, G), lambda i: (i, 0)),
            pl.BlockSpec((NW, 1, 1, UB), lambda i: (0, i, 0, 0)),
            pl.BlockSpec((NW, 1, 1, UB), lambda i: (0, i, 0, 0)),
        ],
        out_specs=(
            pl.BlockSpec((UB, 1), lambda i: (i, 0)),
            pl.BlockSpec((UB, 1), lambda i: (i, 0)),
            pl.BlockSpec((1, G), lambda i: (0, 0)),
        ),
        out_shape=(
            jax.ShapeDtypeStruct((U, 1), jnp.float32),
            jax.ShapeDtypeStruct((U, 1), jnp.float32),
            jax.ShapeDtypeStruct((1, G), jnp.float32),
        ),
    )(la1, la2, p1, p2)


def _tc_normalize(la1, la2, inv1, cs2, flp):
    def body(la1_ref, la2_ref, inv1_ref, cs2_ref, flp_ref,
             lfla1_ref, lfla2_ref, flpT_ref):
        lfla1_ref[...] = jnp.exp(la1_ref[...]) * inv1_ref[...]
        lfla2_ref[...] = jnp.exp(la2_ref[...]) * (1.0 / (cs2_ref[...] + EPS))
        flpT_ref[...] = jnp.transpose(flp_ref[...], (1, 0)).reshape(1, F, UB)

    return pl.pallas_call(
        body,
        grid=(NU,),
        in_specs=[
            pl.BlockSpec((UB, G), lambda i: (i, 0)),
            pl.BlockSpec((UB, G), lambda i: (i, 0)),
            pl.BlockSpec((UB, 1), lambda i: (i, 0)),
            pl.BlockSpec((1, G), lambda i: (0, 0)),
            pl.BlockSpec((UB, F), lambda i: (i, 0)),
        ],
        out_specs=(
            pl.BlockSpec((UB, G), lambda i: (i, 0)),
            pl.BlockSpec((UB, G), lambda i: (i, 0)),
            pl.BlockSpec((1, F, UB), lambda i: (i, 0, 0)),
        ),
        out_shape=(
            jax.ShapeDtypeStruct((U, G), jnp.float32),
            jax.ShapeDtypeStruct((U, G), jnp.float32),
            jax.ShapeDtypeStruct((NU, F, UB), jnp.float32),
        ),
    )(la1, la2, inv1, cs2, flp)


def _tc_reduce(flp, lfla2, fa0p=None):
    def body(*refs):
        if fa0p is not None:
            flp_ref, lfla2_ref, fa0p_ref, g_ref, s_ref, fa0_ref = refs
            fa0_ref[...] = jnp.transpose(
                jnp.sum(fa0p_ref[...].reshape(NW, UB), axis=0, keepdims=True),
                (1, 0))
        else:
            flp_ref, lfla2_ref, g_ref, s_ref = refs
        i = pl.program_id(0)
        gp = lax.dot_general(lfla2_ref[...], flp_ref[...],
                             (((0,), (0,)), ((), ())),
                             preferred_element_type=jnp.float32)
        sp = jnp.sum(flp_ref[...], axis=0, keepdims=True)

        @pl.when(i == 0)
        def _():
            g_ref[...] = gp
            s_ref[...] = sp

        @pl.when(i != 0)
        def _():
            g_ref[...] += gp
            s_ref[...] += sp

    in_specs = [
        pl.BlockSpec((UB, F), lambda i: (i, 0)),
        pl.BlockSpec((UB, G), lambda i: (i, 0)),
    ]
    out_specs = [
        pl.BlockSpec((G, F), lambda i: (0, 0)),
        pl.BlockSpec((1, F), lambda i: (0, 0)),
    ]
    out_shape = [
        jax.ShapeDtypeStruct((G, F), jnp.float32),
        jax.ShapeDtypeStruct((1, F), jnp.float32),
    ]
    args = [flp, lfla2]
    if fa0p is not None:
        in_specs.append(pl.BlockSpec((NW, 1, 1, UB), lambda i: (0, i, 0, 0)))
        out_specs.append(pl.BlockSpec((UB, 1), lambda i: (i, 0)))
        out_shape.append(jax.ShapeDtypeStruct((U, 1), jnp.float32))
        args.append(fa0p.reshape(NW, NU, 1, UB))
    return pl.pallas_call(
        body,
        grid=(NU,),
        in_specs=in_specs,
        out_specs=tuple(out_specs),
        out_shape=tuple(out_shape),
    )(*args)


def _tc_update(flp, hnow, t2p, fa0, sflp, g, lfla1, last):
    def body(flp_ref, hnow_ref, t2p_ref, fa0_ref, s_ref, g_ref, lfla1_ref,
             o_ref, *rest):
        t2p = t2p_ref[...].reshape(2, F, UB)
        t2 = jnp.transpose(t2p[0] + t2p[1], (1, 0))
        mm = lax.dot_general(lfla1_ref[...], g_ref[...],
                             (((1,), (0,)), ((), ())),
                             preferred_element_type=jnp.float32)
        fa0 = fa0_ref[...]
        out = CC * (t2 - fa0 * flp_ref[...] + fa0 * (s_ref[...] / U) + mm) \
            + (1.0 - CC) * hnow_ref[...]
        if last:
            out = out + EPS
            o_ref[...] = out
        else:
            o_ref[...] = out
            rest[0][...] = jnp.transpose(out, (1, 0)).reshape(1, F, UB)

    out_specs = [pl.BlockSpec((UB, F), lambda i: (i, 0))]
    out_shape = [jax.ShapeDtypeStruct((U, F), jnp.float32)]
    if not last:
        out_specs.append(pl.BlockSpec((1, F, UB), lambda i: (i, 0, 0)))
        out_shape.append(jax.ShapeDtypeStruct((NU, F, UB), jnp.float32))

    return pl.pallas_call(
        body,
        grid=(NU,),
        in_specs=[
            pl.BlockSpec((UB, F), lambda i: (i, 0)),
            pl.BlockSpec((UB, F), lambda i: (i, 0)),
            pl.BlockSpec((2, F, 1, 1, UB), lambda i: (0, 0, i, 0, 0)),
            pl.BlockSpec((UB, 1), lambda i: (i, 0)),
            pl.BlockSpec((1, F), lambda i: (0, 0)),
            pl.BlockSpec((G, F), lambda i: (0, 0)),
            pl.BlockSpec((UB, G), lambda i: (i, 0)),
        ],
        out_specs=tuple(out_specs),
        out_shape=tuple(out_shape),
    )(flp, hnow, t2p.reshape(2, F, NU, 1, UB), fa0, sflp, g, lfla1)


def kernel(flp, hnow, w1, w2, la1, la2, r1, r2):
    p1, p2 = _sc_edge_sums(r1, w1, r2, w2)
    inv1, inv2, cs2 = _tc_prep_sums(la1, la2, p1, p2)

    lfw1 = _sc_lfw1(inv1.reshape(U), r1, w1)
    lfw2, fa0p = _sc_lfw2_fa0(inv2.reshape(U), r2, w2, r1, lfw1)

    lfla1, lfla2, flpT = _tc_normalize(la1, la2, inv1, cs2, flp)
    packed = _tc_pack(r1, r2)

    fa0 = None
    for dep in range(DEP):
        if dep == 0:
            g, sflp, fa0 = _tc_reduce(flp, lfla2, fa0p)
        else:
            g, sflp = _tc_reduce(flp, lfla2)
        tp = _sc_spmm(flpT, packed, lfw2, gather_hi=False)
        t2p = _sc_spmm(tp, packed, lfw1, gather_hi=True, paired_src=True)
        last = dep == DEP - 1
        if last:
            (flp,) = _tc_update(flp, hnow, t2p, fa0, sflp, g, lfla1, last=True)
        else:
            flp, flpT = _tc_update(flp, hnow, t2p, fa0, sflp, g, lfla1,
                                   last=False)

    return (lfw1, lfw2, lfla1, lfla2, flp)


# speedup vs baseline: 1.0655x; 1.0655x over previous
import functools

import jax
import jax.numpy as jnp
from jax import lax
from jax.experimental import pallas as pl
from jax.experimental.pallas import tpu as pltpu
from jax.experimental.pallas import tpu_sc as plsc

U = 50000
E = 3200000
F = 16
G = 64
DEP = 2
CC = 0.85
EPS = 1e-16

NC = 2
NS = 16
NW = NC * NS
L = 16

EPT = E // NW
CE = 2000
EPT2 = E // 2
CE2 = 4000

UB = 2000
NU = U // UB


def _sc_params():
    return pltpu.CompilerParams(needs_layout_passes=False,
                                use_tc_tiling_on_sc=False)


def _mesh():
    return plsc.VectorSubcoreMesh(core_axis_name="c", subcore_axis_name="s")


def _wid():
    return lax.axis_index("s") * NC + lax.axis_index("c")


def _sc_edge_sums(r1, w1, r2, w2):
    @functools.partial(
        pl.kernel,
        mesh=_mesh(),
        out_type=(
            jax.ShapeDtypeStruct((NW, U), jnp.float32),
            jax.ShapeDtypeStruct((NW, U), jnp.float32),
        ),
        scratch_types=[
            pltpu.VMEM((U,), jnp.float32),
            pltpu.VMEM((U,), jnp.float32),
            pltpu.VMEM((2, CE), jnp.int32),
            pltpu.VMEM((2, CE), jnp.float32),
            pltpu.VMEM((2, CE), jnp.int32),
            pltpu.VMEM((2, CE), jnp.float32),
            pltpu.SemaphoreType.DMA,
            pltpu.SemaphoreType.DMA,
        ],
        compiler_params=_sc_params(),
    )
    def k(r1_h, w1_h, r2_h, w2_h, p1_h, p2_h, acc1, acc2, i1v, v1v, i2v, v2v,
          sem0, sem1):
        wid = _wid()
        base = wid * EPT
        sems = (sem0, sem1)

        def start(b, chunk):
            off = base + chunk * CE
            pltpu.async_copy(r1_h.at[pl.ds(off, CE)], i1v.at[b], sems[b])
            pltpu.async_copy(w1_h.at[pl.ds(off, CE)], v1v.at[b], sems[b])
            pltpu.async_copy(r2_h.at[pl.ds(off, CE)], i2v.at[b], sems[b])
            pltpu.async_copy(w2_h.at[pl.ds(off, CE)], v2v.at[b], sems[b])

        def drain(b):
            pltpu.make_async_copy(r1_h.at[pl.ds(0, CE)], i1v.at[b], sems[b]).wait()
            pltpu.make_async_copy(w1_h.at[pl.ds(0, CE)], v1v.at[b], sems[b]).wait()
            pltpu.make_async_copy(r2_h.at[pl.ds(0, CE)], i2v.at[b], sems[b]).wait()
            pltpu.make_async_copy(w2_h.at[pl.ds(0, CE)], v2v.at[b], sems[b]).wait()

        start(0, 0)
        start(1, 1)

        @plsc.parallel_loop(0, U // L, unroll=8)
        def _(i):
            z = jnp.zeros((L,), jnp.float32)
            acc1[pl.ds(i * L, L)] = z
            acc2[pl.ds(i * L, L)] = z

        @pl.loop(0, EPT // CE, step=2)
        def _(c):
            for b in range(2):
                drain(b)

                @plsc.parallel_loop(0, CE // L, unroll=8)
                def _(i):
                    sl = pl.ds(i * L, L)
                    plsc.addupdate_scatter(acc1, [i1v[b, sl]],
                                           jnp.exp(v1v[b, sl]))
                    plsc.addupdate_scatter(acc2, [i2v[b, sl]],
                                           jnp.exp(v2v[b, sl]))

                nxt = c + b + 2

                @pl.when(nxt < EPT // CE)
                def _():
                    start(b, nxt)

        pltpu.sync_copy(acc1, p1_h.at[wid])
        pltpu.sync_copy(acc2, p2_h.at[wid])

    return k(r1, w1, r2, w2)


def _sc_lfw1(inv1, r1, w1):
    @functools.partial(
        pl.kernel,
        mesh=_mesh(),
        out_type=jax.ShapeDtypeStruct((E,), jnp.float32),
        scratch_types=[
            pltpu.VMEM((U,), jnp.float32),
            pltpu.VMEM((2, CE), jnp.int32),
            pltpu.VMEM((2, CE), jnp.float32),
            pltpu.VMEM((2, CE), jnp.float32),
            pltpu.SemaphoreType.DMA,
            pltpu.SemaphoreType.DMA,
            pltpu.SemaphoreType.DMA,
            pltpu.SemaphoreType.DMA,
        ],
        compiler_params=_sc_params(),
    )
    def k(inv_h, r_h, w_h, out_h, sv, iv, wv, ov, sem0, sem1, osem0, osem1):
        wid = _wid()
        base = wid * EPT
        sems = (sem0, sem1)
        osems = (osem0, osem1)
        pltpu.sync_copy(inv_h, sv)

        def start(b, chunk):
            off = base + chunk * CE
            pltpu.async_copy(r_h.at[pl.ds(off, CE)], iv.at[b], sems[b])
            pltpu.async_copy(w_h.at[pl.ds(off, CE)], wv.at[b], sems[b])

        def drain(b):
            pltpu.make_async_copy(r_h.at[pl.ds(0, CE)], iv.at[b], sems[b]).wait()
            pltpu.make_async_copy(w_h.at[pl.ds(0, CE)], wv.at[b], sems[b]).wait()

        start(0, 0)
        start(1, 1)

        @pl.loop(0, EPT // CE, step=2)
        def _(c):
            for b in range(2):
                drain(b)
                chunk = c + b

                @pl.when(chunk >= 2)
                def _():
                    pltpu.make_async_copy(ov.at[b], out_h.at[pl.ds(0, CE)],
                                          osems[b]).wait()

                @plsc.parallel_loop(0, CE // L, unroll=8)
                def _(i):
                    sl = pl.ds(i * L, L)
                    d = plsc.load_gather(sv, [iv[b, sl]])
                    ov[b, sl] = jnp.exp(wv[b, sl]) * d

                off = base + chunk * CE
                pltpu.async_copy(ov.at[b], out_h.at[pl.ds(off, CE)], osems[b])
                nxt = chunk + 2

                @pl.when(nxt < EPT // CE)
                def _():
                    start(b, nxt)

        for b in range(2):
            pltpu.make_async_copy(ov.at[b], out_h.at[pl.ds(0, CE)],
                                  osems[b]).wait()

    return k(inv1, r1, w1)


def _sc_lfw2_fa0(inv2, r2, w2, r1, lfw1):
    @functools.partial(
        pl.kernel,
        mesh=_mesh(),
        out_type=(
            jax.ShapeDtypeStruct((E,), jnp.float32),
            jax.ShapeDtypeStruct((NW, U), jnp.float32),
        ),
        scratch_types=[
            pltpu.VMEM((U,), jnp.float32),
            pltpu.VMEM((U,), jnp.float32),
            pltpu.VMEM((2, CE), jnp.int32),
            pltpu.VMEM((2, CE), jnp.float32),
            pltpu.VMEM((2, CE), jnp.int32),
            pltpu.VMEM((2, CE), jnp.float32),
            pltpu.VMEM((2, CE), jnp.float32),
            pltpu.SemaphoreType.DMA,
            pltpu.SemaphoreType.DMA,
            pltpu.SemaphoreType.DMA,
            pltpu.SemaphoreType.DMA,
        ],
        compiler_params=_sc_params(),
    )
    def k(inv_h, r2_h, w2_h, r1_h, lfw1_h, out_h, fp_h,
          sv, facc, i2v, w2v, i1v, l1v, ov, sem0, sem1, osem0, osem1):
        wid = _wid()
        base = wid * EPT
        sems = (sem0, sem1)
        osems = (osem0, osem1)
        pltpu.sync_copy(inv_h, sv)

        def start(b, chunk):
            off = base + chunk * CE
            pltpu.async_copy(r2_h.at[pl.ds(off, CE)], i2v.at[b], sems[b])
            pltpu.async_copy(w2_h.at[pl.ds(off, CE)], w2v.at[b], sems[b])
            pltpu.async_copy(r1_h.at[pl.ds(off, CE)], i1v.at[b], sems[b])
            pltpu.async_copy(lfw1_h.at[pl.ds(off, CE)], l1v.at[b], sems[b])

        def drain(b):
            pltpu.make_async_copy(r2_h.at[pl.ds(0, CE)], i2v.at[b], sems[b]).wait()
            pltpu.make_async_copy(w2_h.at[pl.ds(0, CE)], w2v.at[b], sems[b]).wait()
            pltpu.make_async_copy(r1_h.at[pl.ds(0, CE)], i1v.at[b], sems[b]).wait()
            pltpu.make_async_copy(lfw1_h.at[pl.ds(0, CE)], l1v.at[b], sems[b]).wait()

        start(0, 0)
        start(1, 1)

        @plsc.parallel_loop(0, U // L, unroll=8)
        def _(i):
            facc[pl.ds(i * L, L)] = jnp.zeros((L,), jnp.float32)

        @pl.loop(0, EPT // CE, step=2)
        def _(c):
            for b in range(2):
                drain(b)
                chunk = c + b

                @pl.when(chunk >= 2)
                def _():
                    pltpu.make_async_copy(ov.at[b], out_h.at[pl.ds(0, CE)],
                                          osems[b]).wait()

                @plsc.parallel_loop(0, CE // L, unroll=8)
                def _(i):
                    sl = pl.ds(i * L, L)
                    d = plsc.load_gather(sv, [i2v[b, sl]])
                    o = jnp.exp(w2v[b, sl]) * d
                    ov[b, sl] = o
                    plsc.addupdate_scatter(facc, [i1v[b, sl]], o * l1v[b, sl])

                off = base + chunk * CE
                pltpu.async_copy(ov.at[b], out_h.at[pl.ds(off, CE)], osems[b])
                nxt = chunk + 2

                @pl.when(nxt < EPT // CE)
                def _():
                    start(b, nxt)

        for b in range(2):
            pltpu.make_async_copy(ov.at[b], out_h.at[pl.ds(0, CE)],
                                  osems[b]).wait()
        pltpu.sync_copy(facc, fp_h.at[wid])

    return k(inv2, r2, w2, r1, lfw1)


def _sc_spmm(srcT, packed, wgt, gather_hi, paired_src=False):
    NCH = EPT2 // CE2

    scratch = [
        pltpu.VMEM((U,), jnp.float32),
        pltpu.VMEM((U,), jnp.float32),
        pltpu.VMEM((CE2,), jnp.int32),
        pltpu.VMEM((CE2,), jnp.float32),
        pltpu.VMEM((CE2,), jnp.int32),
        pltpu.VMEM((CE2,), jnp.float32),
        pltpu.SemaphoreType.DMA,
        pltpu.SemaphoreType.DMA,
    ]
    if paired_src:
        scratch.insert(2, pltpu.VMEM((UB,), jnp.float32))

    @functools.partial(
        pl.kernel,
        mesh=_mesh(),
        out_type=jax.ShapeDtypeStruct((2, F, U), jnp.float32),
        scratch_types=scratch,
        compiler_params=_sc_params(),
    )
    def k(srcT_h, pk_h, w_h, out_h, col, acc, *rest):
        if paired_src:
            tmp, pk0, w0, pk1, w1, sem0, sem1 = rest
        else:
            pk0, w0, pk1, w1, sem0, sem1 = rest
        wid = _wid()
        d = wid % F
        g = wid // F
        base = g * EPT2
        bufs = ((pk0, w0, sem0), (pk1, w1, sem1))

        def start(b, chunk):
            pkb, wb, semb = bufs[b]
            off = base + chunk * CE2
            pltpu.async_copy(pk_h.at[pl.ds(off, CE2)], pkb, semb)
            pltpu.async_copy(w_h.at[pl.ds(off, CE2)], wb, semb)

        def drain(b):
            pkb, wb, semb = bufs[b]
            pltpu.make_async_copy(pk_h.at[pl.ds(0, CE2)], pkb, semb).wait()
            pltpu.make_async_copy(w_h.at[pl.ds(0, CE2)], wb, semb).wait()

        start(0, 0)
        start(1, 1)

        if paired_src:
            @pl.loop(0, NU)
            def _(j):
                pltpu.sync_copy(srcT_h.at[0, d, pl.ds(j * UB, UB)],
                                col.at[pl.ds(j * UB, UB)])
                pltpu.sync_copy(srcT_h.at[1, d, pl.ds(j * UB, UB)], tmp)

                @plsc.parallel_loop(0, UB // L, unroll=8)
                def _(i):
                    sl = pl.ds(j * UB + i * L, L)
                    col[sl] = col[sl] + tmp[pl.ds(i * L, L)]
        else:
            @pl.loop(0, NU)
            def _(j):
                pltpu.sync_copy(srcT_h.at[j, d], col.at[pl.ds(j * UB, UB)])

        @plsc.parallel_loop(0, U // L, unroll=8)
        def _(i):
            acc[pl.ds(i * L, L)] = jnp.zeros((L,), jnp.float32)

        @pl.loop(0, NCH, step=2)
        def _(c):
            for b in range(2):
                pkb, wb, _ = bufs[b]
                drain(b)

                @plsc.parallel_loop(0, CE2 // L, unroll=16)
                def _(i):
                    sl = pl.ds(i * L, L)
                    pk = pkb[sl]
                    hi = lax.shift_right_logical(pk, 16)
                    lo = lax.bitwise_and(pk, 0xFFFF)
                    if gather_hi:
                        v = plsc.load_gather(col, [lo])
                        plsc.addupdate_scatter(acc, [hi], v * wb[sl])
                    else:
                        v = plsc.load_gather(col, [hi])
                        plsc.addupdate_scatter(acc, [lo], v * wb[sl])

                nxt = c + b + 2

                @pl.when(nxt < NCH)
                def _():
                    start(b, nxt)

        pltpu.sync_copy(acc, out_h.at[g, d])

    return k(srcT, packed, wgt)


def _tc_prep_sums(la1, la2, p1, p2):
    def body(la1_ref, la2_ref, p1_ref, p2_ref, inv1_ref, inv2_ref, cs2_ref):
        i = pl.program_id(0)
        e1 = jnp.exp(la1_ref[...])
        rs = jnp.sum(e1, axis=1, keepdims=True)
        p1b = p1_ref[...].reshape(NW, UB)
        p2b = p2_ref[...].reshape(NW, UB)
        ps1 = jnp.transpose(jnp.sum(p1b, axis=0, keepdims=True), (1, 0))
        ps2 = jnp.transpose(jnp.sum(p2b, axis=0, keepdims=True), (1, 0))
        inv1_ref[...] = 1.0 / (rs + EPS + ps1)
        inv2_ref[...] = 1.0 / (EPS + ps2)
        part = jnp.sum(jnp.exp(la2_ref[...]), axis=0, keepdims=True)

        @pl.when(i == 0)
        def _():
            cs2_ref[...] = part

        @pl.when(i != 0)
        def _():
            cs2_ref[...] += part

    p1 = p1.reshape(NW, NU, 1, UB)
    p2 = p2.reshape(NW, NU, 1, UB)
    return pl.pallas_call(
        body,
        grid=(NU,),
        in_specs=[
            pl.BlockSpec((UB, G), lambda i: (i, 0)),
            pl.BlockSpec((UB, G), lambda i: (i, 0)),
            pl.BlockSpec((NW, 1, 1, UB), lambda i: (0, i, 0, 0)),
            pl.BlockSpec((NW, 1, 1, UB), lambda i: (0, i, 0, 0)),
        ],
        out_specs=(
            pl.BlockSpec((UB, 1), lambda i: (i, 0)),
            pl.BlockSpec((UB, 1), lambda i: (i, 0)),
            pl.BlockSpec((1, G), lambda i: (0, 0)),
        ),
        out_shape=(
            jax.ShapeDtypeStruct((U, 1), jnp.float32),
            jax.ShapeDtypeStruct((U, 1), jnp.float32),
            jax.ShapeDtypeStruct((1, G), jnp.float32),
        ),
    )(la1, la2, p1, p2)


def _tc_normalize(la1, la2, inv1, cs2, flp, r1, r2, fa0p):
    def body(la1_ref, la2_ref, inv1_ref, cs2_ref, flp_ref, r1_ref, r2_ref,
             fa0p_ref, lfla1_ref, lfla2_ref, flpT_ref, pk_ref, g_ref, s_ref,
             fa0_ref):
        i = pl.program_id(0)
        flp = flp_ref[...]
        lfla2 = jnp.exp(la2_ref[...]) * (1.0 / (cs2_ref[...] + EPS))
        lfla1_ref[...] = jnp.exp(la1_ref[...]) * inv1_ref[...]
        lfla2_ref[...] = lfla2
        flpT_ref[...] = jnp.transpose(flp, (1, 0)).reshape(1, F, UB)
        pk_ref[...] = lax.bitwise_or(lax.shift_left(r1_ref[...], 16),
                                     r2_ref[...])
        fa0_ref[...] = jnp.transpose(
            jnp.sum(fa0p_ref[...].reshape(NW, UB), axis=0, keepdims=True),
            (1, 0))
        gp = lax.dot_general(lfla2, flp, (((0,), (0,)), ((), ())),
                             preferred_element_type=jnp.float32)
        sp = jnp.sum(flp, axis=0, keepdims=True)

        @pl.when(i == 0)
        def _():
            g_ref[...] = gp
            s_ref[...] = sp

        @pl.when(i != 0)
        def _():
            g_ref[...] += gp
            s_ref[...] += sp

    lfla1, lfla2, flpT, packed, g, sflp, fa0 = pl.pallas_call(
        body,
        grid=(NU,),
        in_specs=[
            pl.BlockSpec((UB, G), lambda i: (i, 0)),
            pl.BlockSpec((UB, G), lambda i: (i, 0)),
            pl.BlockSpec((UB, 1), lambda i: (i, 0)),
            pl.BlockSpec((1, G), lambda i: (0, 0)),
            pl.BlockSpec((UB, F), lambda i: (i, 0)),
            pl.BlockSpec((UB, G), lambda i: (i, 0)),
            pl.BlockSpec((UB, G), lambda i: (i, 0)),
            pl.BlockSpec((NW, 1, 1, UB), lambda i: (0, i, 0, 0)),
        ],
        out_specs=(
            pl.BlockSpec((UB, G), lambda i: (i, 0)),
            pl.BlockSpec((UB, G), lambda i: (i, 0)),
            pl.BlockSpec((1, F, UB), lambda i: (i, 0, 0)),
            pl.BlockSpec((UB, G), lambda i: (i, 0)),
            pl.BlockSpec((G, F), lambda i: (0, 0)),
            pl.BlockSpec((1, F), lambda i: (0, 0)),
            pl.BlockSpec((UB, 1), lambda i: (i, 0)),
        ),
        out_shape=(
            jax.ShapeDtypeStruct((U, G), jnp.float32),
            jax.ShapeDtypeStruct((U, G), jnp.float32),
            jax.ShapeDtypeStruct((NU, F, UB), jnp.float32),
            jax.ShapeDtypeStruct((U, G), jnp.int32),
            jax.ShapeDtypeStruct((G, F), jnp.float32),
            jax.ShapeDtypeStruct((1, F), jnp.float32),
            jax.ShapeDtypeStruct((U, 1), jnp.float32),
        ),
    )(la1, la2, inv1, cs2, flp, r1.reshape(U, G), r2.reshape(U, G),
      fa0p.reshape(NW, NU, 1, UB))
    return lfla1, lfla2, flpT, packed.reshape(E), g, sflp, fa0


def _tc_update(flp, hnow, t2p, fa0, sflp, g, lfla1, last, lfla2=None):
    def body(flp_ref, hnow_ref, t2p_ref, fa0_ref, s_ref, g_ref, lfla1_ref,
             *rest):
        t2p = t2p_ref[...].reshape(2, F, UB)
        t2 = jnp.transpose(t2p[0] + t2p[1], (1, 0))
        mm = lax.dot_general(lfla1_ref[...], g_ref[...],
                             (((1,), (0,)), ((), ())),
                             preferred_element_type=jnp.float32)
        fa0 = fa0_ref[...]
        out = CC * (t2 - fa0 * flp_ref[...] + fa0 * (s_ref[...] / U) + mm) \
            + (1.0 - CC) * hnow_ref[...]
        if last:
            (o_ref,) = rest
            o_ref[...] = out + EPS
        else:
            lfla2_ref, o_ref, oT_ref, gn_ref, sn_ref = rest
            o_ref[...] = out
            oT_ref[...] = jnp.transpose(out, (1, 0)).reshape(1, F, UB)
            gp = lax.dot_general(lfla2_ref[...], out, (((0,), (0,)), ((), ())),
                                 preferred_element_type=jnp.float32)
            sp = jnp.sum(out, axis=0, keepdims=True)
            i = pl.program_id(0)

            @pl.when(i == 0)
            def _():
                gn_ref[...] = gp
                sn_ref[...] = sp

            @pl.when(i != 0)
            def _():
                gn_ref[...] += gp
                sn_ref[...] += sp

    in_specs = [
        pl.BlockSpec((UB, F), lambda i: (i, 0)),
        pl.BlockSpec((UB, F), lambda i: (i, 0)),
        pl.BlockSpec((2, F, 1, 1, UB), lambda i: (0, 0, i, 0, 0)),
        pl.BlockSpec((UB, 1), lambda i: (i, 0)),
        pl.BlockSpec((1, F), lambda i: (0, 0)),
        pl.BlockSpec((G, F), lambda i: (0, 0)),
        pl.BlockSpec((UB, G), lambda i: (i, 0)),
    ]
    args = [flp, hnow, t2p.reshape(2, F, NU, 1, UB), fa0, sflp, g, lfla1]
    out_specs = [pl.BlockSpec((UB, F), lambda i: (i, 0))]
    out_shape = [jax.ShapeDtypeStruct((U, F), jnp.float32)]
    if not last:
        in_specs.append(pl.BlockSpec((UB, G), lambda i: (i, 0)))
        args.append(lfla2)
        out_specs += [
            pl.BlockSpec((1, F, UB), lambda i: (i, 0, 0)),
            pl.BlockSpec((G, F), lambda i: (0, 0)),
            pl.BlockSpec((1, F), lambda i: (0, 0)),
        ]
        out_shape += [
            jax.ShapeDtypeStruct((NU, F, UB), jnp.float32),
            jax.ShapeDtypeStruct((G, F), jnp.float32),
            jax.ShapeDtypeStruct((1, F), jnp.float32),
        ]

    return pl.pallas_call(
        body,
        grid=(NU,),
        in_specs=in_specs,
        out_specs=tuple(out_specs),
        out_shape=tuple(out_shape),
    )(*args)


def kernel(flp, hnow, w1, w2, la1, la2, r1, r2):
    p1, p2 = _sc_edge_sums(r1, w1, r2, w2)
    inv1, inv2, cs2 = _tc_prep_sums(la1, la2, p1, p2)

    lfw1 = _sc_lfw1(inv1.reshape(U), r1, w1)
    lfw2, fa0p = _sc_lfw2_fa0(inv2.reshape(U), r2, w2, r1, lfw1)

    lfla1, lfla2, flpT, packed, g, sflp, fa0 = _tc_normalize(
        la1, la2, inv1, cs2, flp, r1, r2, fa0p)

    for dep in range(DEP):
        tp = _sc_spmm(flpT, packed, lfw2, gather_hi=False)
        t2p = _sc_spmm(tp, packed, lfw1, gather_hi=True, paired_src=True)
        last = dep == DEP - 1
        if last:
            (flp,) = _tc_update(flp, hnow, t2p, fa0, sflp, g, lfla1, last=True)
        else:
            flp, flpT, g, sflp = _tc_update(flp, hnow, t2p, fa0, sflp, g,
                                            lfla1, last=False, lfla2=lfla2)

    return (lfw1, lfw2, lfla1, lfla2, flp)


# speedup vs baseline: 1.0758x; 1.0097x over previous
import functools

import jax
import jax.numpy as jnp
from jax import lax
from jax.experimental import pallas as pl
from jax.experimental.pallas import tpu as pltpu
from jax.experimental.pallas import tpu_sc as plsc

U = 50000
E = 3200000
F = 16
G = 64
DEP = 2
CC = 0.85
EPS = 1e-16

NC = 2
NS = 16
NW = NC * NS
L = 16

EPT = E // NW
CE = 2000
EPT2 = E // 2
CE2 = 4000

UB = 2000
NU = U // UB


def _sc_params():
    return pltpu.CompilerParams(needs_layout_passes=False,
                                use_tc_tiling_on_sc=False)


def _mesh():
    return plsc.VectorSubcoreMesh(core_axis_name="c", subcore_axis_name="s")


def _wid():
    return lax.axis_index("s") * NC + lax.axis_index("c")


def _sc_edge_sums(r1, w1, r2, w2):
    @functools.partial(
        pl.kernel,
        mesh=_mesh(),
        out_type=(
            jax.ShapeDtypeStruct((NW, U), jnp.float32),
            jax.ShapeDtypeStruct((NW, U), jnp.float32),
        ),
        scratch_types=[
            pltpu.VMEM((U,), jnp.float32),
            pltpu.VMEM((U,), jnp.float32),
            pltpu.VMEM((2, CE), jnp.int32),
            pltpu.VMEM((2, CE), jnp.float32),
            pltpu.VMEM((2, CE), jnp.int32),
            pltpu.VMEM((2, CE), jnp.float32),
            pltpu.SemaphoreType.DMA,
            pltpu.SemaphoreType.DMA,
        ],
        compiler_params=_sc_params(),
    )
    def k(r1_h, w1_h, r2_h, w2_h, p1_h, p2_h, acc1, acc2, i1v, v1v, i2v, v2v,
          sem0, sem1):
        wid = _wid()
        base = wid * EPT
        sems = (sem0, sem1)

        def start(b, chunk):
            off = base + chunk * CE
            pltpu.async_copy(r1_h.at[pl.ds(off, CE)], i1v.at[b], sems[b])
            pltpu.async_copy(w1_h.at[pl.ds(off, CE)], v1v.at[b], sems[b])
            pltpu.async_copy(r2_h.at[pl.ds(off, CE)], i2v.at[b], sems[b])
            pltpu.async_copy(w2_h.at[pl.ds(off, CE)], v2v.at[b], sems[b])

        def drain(b):
            pltpu.make_async_copy(r1_h.at[pl.ds(0, CE)], i1v.at[b], sems[b]).wait()
            pltpu.make_async_copy(w1_h.at[pl.ds(0, CE)], v1v.at[b], sems[b]).wait()
            pltpu.make_async_copy(r2_h.at[pl.ds(0, CE)], i2v.at[b], sems[b]).wait()
            pltpu.make_async_copy(w2_h.at[pl.ds(0, CE)], v2v.at[b], sems[b]).wait()

        start(0, 0)
        start(1, 1)

        @plsc.parallel_loop(0, U // L, unroll=8)
        def _(i):
            z = jnp.zeros((L,), jnp.float32)
            acc1[pl.ds(i * L, L)] = z
            acc2[pl.ds(i * L, L)] = z

        @pl.loop(0, EPT // CE, step=2)
        def _(c):
            for b in range(2):
                drain(b)

                @plsc.parallel_loop(0, CE // L, unroll=8)
                def _(i):
                    sl = pl.ds(i * L, L)
                    plsc.addupdate_scatter(acc1, [i1v[b, sl]],
                                           jnp.exp(v1v[b, sl]))
                    plsc.addupdate_scatter(acc2, [i2v[b, sl]],
                                           jnp.exp(v2v[b, sl]))

                nxt = c + b + 2

                @pl.when(nxt < EPT // CE)
                def _():
                    start(b, nxt)

        pltpu.sync_copy(acc1, p1_h.at[wid])
        pltpu.sync_copy(acc2, p2_h.at[wid])

    return k(r1, w1, r2, w2)


def _sc_lfw1(inv1, r1, w1):
    @functools.partial(
        pl.kernel,
        mesh=_mesh(),
        out_type=jax.ShapeDtypeStruct((E,), jnp.float32),
        scratch_types=[
            pltpu.VMEM((U,), jnp.float32),
            pltpu.VMEM((2, CE), jnp.int32),
            pltpu.VMEM((2, CE), jnp.float32),
            pltpu.VMEM((2, CE), jnp.float32),
            pltpu.SemaphoreType.DMA,
            pltpu.SemaphoreType.DMA,
            pltpu.SemaphoreType.DMA,
            pltpu.SemaphoreType.DMA,
        ],
        compiler_params=_sc_params(),
    )
    def k(inv_h, r_h, w_h, out_h, sv, iv, wv, ov, sem0, sem1, osem0, osem1):
        wid = _wid()
        base = wid * EPT
        sems = (sem0, sem1)
        osems = (osem0, osem1)
        pltpu.sync_copy(inv_h, sv)

        def start(b, chunk):
            off = base + chunk * CE
            pltpu.async_copy(r_h.at[pl.ds(off, CE)], iv.at[b], sems[b])
            pltpu.async_copy(w_h.at[pl.ds(off, CE)], wv.at[b], sems[b])

        def drain(b):
            pltpu.make_async_copy(r_h.at[pl.ds(0, CE)], iv.at[b], sems[b]).wait()
            pltpu.make_async_copy(w_h.at[pl.ds(0, CE)], wv.at[b], sems[b]).wait()

        start(0, 0)
        start(1, 1)

        @pl.loop(0, EPT // CE, step=2)
        def _(c):
            for b in range(2):
                drain(b)
                chunk = c + b

                @pl.when(chunk >= 2)
                def _():
                    pltpu.make_async_copy(ov.at[b], out_h.at[pl.ds(0, CE)],
                                          osems[b]).wait()

                @plsc.parallel_loop(0, CE // L, unroll=8)
                def _(i):
                    sl = pl.ds(i * L, L)
                    d = plsc.load_gather(sv, [iv[b, sl]])
                    ov[b, sl] = jnp.exp(wv[b, sl]) * d

                off = base + chunk * CE
                pltpu.async_copy(ov.at[b], out_h.at[pl.ds(off, CE)], osems[b])
                nxt = chunk + 2

                @pl.when(nxt < EPT // CE)
                def _():
                    start(b, nxt)

        for b in range(2):
            pltpu.make_async_copy(ov.at[b], out_h.at[pl.ds(0, CE)],
                                  osems[b]).wait()

    return k(inv1, r1, w1)


def _sc_lfw2_fa0(inv2, r2, w2, r1, lfw1):
    @functools.partial(
        pl.kernel,
        mesh=_mesh(),
        out_type=(
            jax.ShapeDtypeStruct((E,), jnp.float32),
            jax.ShapeDtypeStruct((NW, U), jnp.float32),
        ),
        scratch_types=[
            pltpu.VMEM((U,), jnp.float32),
            pltpu.VMEM((U,), jnp.float32),
            pltpu.VMEM((2, CE), jnp.int32),
            pltpu.VMEM((2, CE), jnp.float32),
            pltpu.VMEM((2, CE), jnp.int32),
            pltpu.VMEM((2, CE), jnp.float32),
            pltpu.VMEM((2, CE), jnp.float32),
            pltpu.SemaphoreType.DMA,
            pltpu.SemaphoreType.DMA,
            pltpu.SemaphoreType.DMA,
            pltpu.SemaphoreType.DMA,
        ],
        compiler_params=_sc_params(),
    )
    def k(inv_h, r2_h, w2_h, r1_h, lfw1_h, out_h, fp_h,
          sv, facc, i2v, w2v, i1v, l1v, ov, sem0, sem1, osem0, osem1):
        wid = _wid()
        base = wid * EPT
        sems = (sem0, sem1)
        osems = (osem0, osem1)
        pltpu.sync_copy(inv_h, sv)

        def start(b, chunk):
            off = base + chunk * CE
            pltpu.async_copy(r2_h.at[pl.ds(off, CE)], i2v.at[b], sems[b])
            pltpu.async_copy(w2_h.at[pl.ds(off, CE)], w2v.at[b], sems[b])
            pltpu.async_copy(r1_h.at[pl.ds(off, CE)], i1v.at[b], sems[b])
            pltpu.async_copy(lfw1_h.at[pl.ds(off, CE)], l1v.at[b], sems[b])

        def drain(b):
            pltpu.make_async_copy(r2_h.at[pl.ds(0, CE)], i2v.at[b], sems[b]).wait()
            pltpu.make_async_copy(w2_h.at[pl.ds(0, CE)], w2v.at[b], sems[b]).wait()
            pltpu.make_async_copy(r1_h.at[pl.ds(0, CE)], i1v.at[b], sems[b]).wait()
            pltpu.make_async_copy(lfw1_h.at[pl.ds(0, CE)], l1v.at[b], sems[b]).wait()

        start(0, 0)
        start(1, 1)

        @plsc.parallel_loop(0, U // L, unroll=8)
        def _(i):
            facc[pl.ds(i * L, L)] = jnp.zeros((L,), jnp.float32)

        @pl.loop(0, EPT // CE, step=2)
        def _(c):
            for b in range(2):
                drain(b)
                chunk = c + b

                @pl.when(chunk >= 2)
                def _():
                    pltpu.make_async_copy(ov.at[b], out_h.at[pl.ds(0, CE)],
                                          osems[b]).wait()

                @plsc.parallel_loop(0, CE // L, unroll=8)
                def _(i):
                    sl = pl.ds(i * L, L)
                    d = plsc.load_gather(sv, [i2v[b, sl]])
                    o = jnp.exp(w2v[b, sl]) * d
                    ov[b, sl] = o
                    plsc.addupdate_scatter(facc, [i1v[b, sl]], o * l1v[b, sl])

                off = base + chunk * CE
                pltpu.async_copy(ov.at[b], out_h.at[pl.ds(off, CE)], osems[b])
                nxt = chunk + 2

                @pl.when(nxt < EPT // CE)
                def _():
                    start(b, nxt)

        for b in range(2):
            pltpu.make_async_copy(ov.at[b], out_h.at[pl.ds(0, CE)],
                                  osems[b]).wait()
        pltpu.sync_copy(facc, fp_h.at[wid])

    return k(inv2, r2, w2, r1, lfw1)


def _sc_spmm(srcT, packed, wgt, gather_hi, paired_src=False):
    NCH = EPT2 // CE2

    scratch = [
        pltpu.VMEM((U,), jnp.float32),
        pltpu.VMEM((U,), jnp.float32),
        pltpu.VMEM((CE2,), jnp.int32),
        pltpu.VMEM((CE2,), jnp.float32),
        pltpu.VMEM((CE2,), jnp.int32),
        pltpu.VMEM((CE2,), jnp.float32),
        pltpu.SemaphoreType.DMA,
        pltpu.SemaphoreType.DMA,
    ]
    if paired_src:
        scratch.insert(2, pltpu.VMEM((UB,), jnp.float32))

    @functools.partial(
        pl.kernel,
        mesh=_mesh(),
        out_type=jax.ShapeDtypeStruct((2, F, U), jnp.float32),
        scratch_types=scratch,
        compiler_params=_sc_params(),
    )
    def k(srcT_h, pk_h, w_h, out_h, col, acc, *rest):
        if paired_src:
            tmp, pk0, w0, pk1, w1, sem0, sem1 = rest
        else:
            pk0, w0, pk1, w1, sem0, sem1 = rest
        wid = _wid()
        d = wid % F
        g = wid // F
        base = g * EPT2
        bufs = ((pk0, w0, sem0), (pk1, w1, sem1))

        def start(b, chunk):
            pkb, wb, semb = bufs[b]
            off = base + chunk * CE2
            pltpu.async_copy(pk_h.at[pl.ds(off, CE2)], pkb, semb)
            pltpu.async_copy(w_h.at[pl.ds(off, CE2)], wb, semb)

        def drain(b):
            pkb, wb, semb = bufs[b]
            pltpu.make_async_copy(pk_h.at[pl.ds(0, CE2)], pkb, semb).wait()
            pltpu.make_async_copy(w_h.at[pl.ds(0, CE2)], wb, semb).wait()

        start(0, 0)
        start(1, 1)

        if paired_src:
            @pl.loop(0, NU)
            def _(j):
                pltpu.sync_copy(srcT_h.at[0, d, pl.ds(j * UB, UB)],
                                col.at[pl.ds(j * UB, UB)])
                pltpu.sync_copy(srcT_h.at[1, d, pl.ds(j * UB, UB)], tmp)

                @plsc.parallel_loop(0, UB // L, unroll=8)
                def _(i):
                    sl = pl.ds(j * UB + i * L, L)
                    col[sl] = col[sl] + tmp[pl.ds(i * L, L)]
        else:
            @pl.loop(0, NU)
            def _(j):
                pltpu.sync_copy(srcT_h.at[j, d], col.at[pl.ds(j * UB, UB)])

        @plsc.parallel_loop(0, U // L, unroll=8)
        def _(i):
            acc[pl.ds(i * L, L)] = jnp.zeros((L,), jnp.float32)

        @pl.loop(0, NCH, step=2)
        def _(c):
            for b in range(2):
                pkb, wb, _ = bufs[b]
                drain(b)

                @plsc.parallel_loop(0, CE2 // L, unroll=16)
                def _(i):
                    sl = pl.ds(i * L, L)
                    pk = pkb[sl]
                    hi = lax.shift_right_logical(pk, 16)
                    lo = lax.bitwise_and(pk, 0xFFFF)
                    if gather_hi:
                        v = plsc.load_gather(col, [lo])
                        plsc.addupdate_scatter(acc, [hi], v * wb[sl])
                    else:
                        v = plsc.load_gather(col, [hi])
                        plsc.addupdate_scatter(acc, [lo], v * wb[sl])

                nxt = c + b + 2

                @pl.when(nxt < NCH)
                def _():
                    start(b, nxt)

        pltpu.sync_copy(acc, out_h.at[g, d])

    return k(srcT, packed, wgt)


def _tc_prep_sums(la1, la2, p1, p2):
    def body(la1_ref, la2_ref, p1_ref, p2_ref, inv1_ref, inv2_ref, cs2_ref):
        i = pl.program_id(0)
        e1 = jnp.exp(la1_ref[...])
        rs = jnp.sum(e1, axis=1, keepdims=True)
        p1b = p1_ref[...].reshape(NW, UB)
        p2b = p2_ref[...].reshape(NW, UB)
        ps1 = jnp.transpose(jnp.sum(p1b, axis=0, keepdims=True), (1, 0))
        ps2 = jnp.transpose(jnp.sum(p2b, axis=0, keepdims=True), (1, 0))
        inv1_ref[...] = 1.0 / (rs + EPS + ps1)
        inv2_ref[...] = 1.0 / (EPS + ps2)
        part = jnp.sum(jnp.exp(la2_ref[...]), axis=0, keepdims=True)

        @pl.when(i == 0)
        def _():
            cs2_ref[...] = part

        @pl.when(i != 0)
        def _():
            cs2_ref[...] += part

    p1 = p1.reshape(NW, NU, 1, UB)
    p2 = p2.reshape(NW, NU, 1, UB)
    return pl.pallas_call(
        body,
        grid=(NU,),
        in_specs=[
            pl.BlockSpec((UB, G), lambda i: (i, 0)),
            pl.BlockSpec((UB, G), lambda i: (i, 0)),
            pl.BlockSpec((NW, 1, 1, UB), lambda i: (0, i, 0, 0)),
            pl.BlockSpec((NW, 1, 1, UB), lambda i: (0, i, 0, 0)),
        ],
        out_specs=(
            pl.BlockSpec((UB, 1), lambda i: (i, 0)),
            pl.BlockSpec((UB, 1), lambda i: (i, 0)),
            pl.BlockSpec((1, G), lambda i: (0, 0)),
        ),
        out_shape=(
            jax.ShapeDtypeStruct((U, 1), jnp.float32),
            jax.ShapeDtypeStruct((U, 1), jnp.float32),
            jax.ShapeDtypeStruct((1, G), jnp.float32),
        ),
    )(la1, la2, p1, p2)


def _tc_pack(r1, r2):
    r1m = r1.reshape(U, G)
    r2m = r2.reshape(U, G)

    def body(a_ref, b_ref, o_ref):
        o_ref[...] = lax.bitwise_or(lax.shift_left(a_ref[...], 16), b_ref[...])

    out = pl.pallas_call(
        body,
        grid=(NU,),
        in_specs=[
            pl.BlockSpec((UB, G), lambda i: (i, 0)),
            pl.BlockSpec((UB, G), lambda i: (i, 0)),
        ],
        out_specs=pl.BlockSpec((UB, G), lambda i: (i, 0)),
        out_shape=jax.ShapeDtypeStruct((U, G), jnp.int32),
    )(r1m, r2m)
    return out.reshape(E)


def _tc_normalize(la1, la2, inv1, cs2, flp):
    def body(la1_ref, la2_ref, inv1_ref, cs2_ref, flp_ref,
             lfla1_ref, lfla2_ref, flpT_ref):
        lfla1_ref[...] = jnp.exp(la1_ref[...]) * inv1_ref[...]
        lfla2_ref[...] = jnp.exp(la2_ref[...]) * (1.0 / (cs2_ref[...] + EPS))
        flpT_ref[...] = jnp.transpose(flp_ref[...], (1, 0)).reshape(1, F, UB)

    return pl.pallas_call(
        body,
        grid=(NU,),
        in_specs=[
            pl.BlockSpec((UB, G), lambda i: (i, 0)),
            pl.BlockSpec((UB, G), lambda i: (i, 0)),
            pl.BlockSpec((UB, 1), lambda i: (i, 0)),
            pl.BlockSpec((1, G), lambda i: (0, 0)),
            pl.BlockSpec((UB, F), lambda i: (i, 0)),
        ],
        out_specs=(
            pl.BlockSpec((UB, G), lambda i: (i, 0)),
            pl.BlockSpec((UB, G), lambda i: (i, 0)),
            pl.BlockSpec((1, F, UB), lambda i: (i, 0, 0)),
        ),
        out_shape=(
            jax.ShapeDtypeStruct((U, G), jnp.float32),
            jax.ShapeDtypeStruct((U, G), jnp.float32),
            jax.ShapeDtypeStruct((NU, F, UB), jnp.float32),
        ),
    )(la1, la2, inv1, cs2, flp)


def _tc_reduce(flp, lfla2, fa0p=None):
    def body(*refs):
        if fa0p is not None:
            flp_ref, lfla2_ref, fa0p_ref, g_ref, s_ref, fa0_ref = refs
            fa0_ref[...] = jnp.transpose(
                jnp.sum(fa0p_ref[...].reshape(NW, UB), axis=0, keepdims=True),
                (1, 0))
        else:
            flp_ref, lfla2_ref, g_ref, s_ref = refs
        i = pl.program_id(0)
        gp = lax.dot_general(lfla2_ref[...], flp_ref[...],
                             (((0,), (0,)), ((), ())),
                             preferred_element_type=jnp.float32)
        sp = jnp.sum(flp_ref[...], axis=0, keepdims=True)

        @pl.when(i == 0)
        def _():
            g_ref[...] = gp
            s_ref[...] = sp

        @pl.when(i != 0)
        def _():
            g_ref[...] += gp
            s_ref[...] += sp

    in_specs = [
        pl.BlockSpec((UB, F), lambda i: (i, 0)),
        pl.BlockSpec((UB, G), lambda i: (i, 0)),
    ]
    out_specs = [
        pl.BlockSpec((G, F), lambda i: (0, 0)),
        pl.BlockSpec((1, F), lambda i: (0, 0)),
    ]
    out_shape = [
        jax.ShapeDtypeStruct((G, F), jnp.float32),
        jax.ShapeDtypeStruct((1, F), jnp.float32),
    ]
    args = [flp, lfla2]
    if fa0p is not None:
        in_specs.append(pl.BlockSpec((NW, 1, 1, UB), lambda i: (0, i, 0, 0)))
        out_specs.append(pl.BlockSpec((UB, 1), lambda i: (i, 0)))
        out_shape.append(jax.ShapeDtypeStruct((U, 1), jnp.float32))
        args.append(fa0p.reshape(NW, NU, 1, UB))
    return pl.pallas_call(
        body,
        grid=(NU,),
        in_specs=in_specs,
        out_specs=tuple(out_specs),
        out_shape=tuple(out_shape),
    )(*args)


def _tc_update(flp, hnow, t2p, fa0, sflp, g, lfla1, last):
    def body(flp_ref, hnow_ref, t2p_ref, fa0_ref, s_ref, g_ref, lfla1_ref,
             o_ref, *rest):
        t2p = t2p_ref[...].reshape(2, F, UB)
        t2 = jnp.transpose(t2p[0] + t2p[1], (1, 0))
        mm = lax.dot_general(lfla1_ref[...], g_ref[...],
                             (((1,), (0,)), ((), ())),
                             preferred_element_type=jnp.float32)
        fa0 = fa0_ref[...]
        out = CC * (t2 - fa0 * flp_ref[...] + fa0 * (s_ref[...] / U) + mm) \
            + (1.0 - CC) * hnow_ref[...]
        if last:
            out = out + EPS
            o_ref[...] = out
        else:
            o_ref[...] = out
            rest[0][...] = jnp.transpose(out, (1, 0)).reshape(1, F, UB)

    out_specs = [pl.BlockSpec((UB, F), lambda i: (i, 0))]
    out_shape = [jax.ShapeDtypeStruct((U, F), jnp.float32)]
    if not last:
        out_specs.append(pl.BlockSpec((1, F, UB), lambda i: (i, 0, 0)))
        out_shape.append(jax.ShapeDtypeStruct((NU, F, UB), jnp.float32))

    return pl.pallas_call(
        body,
        grid=(NU,),
        in_specs=[
            pl.BlockSpec((UB, F), lambda i: (i, 0)),
            pl.BlockSpec((UB, F), lambda i: (i, 0)),
            pl.BlockSpec((2, F, 1, 1, UB), lambda i: (0, 0, i, 0, 0)),
            pl.BlockSpec((UB, 1), lambda i: (i, 0)),
            pl.BlockSpec((1, F), lambda i: (0, 0)),
            pl.BlockSpec((G, F), lambda i: (0, 0)),
            pl.BlockSpec((UB, G), lambda i: (i, 0)),
        ],
        out_specs=tuple(out_specs),
        out_shape=tuple(out_shape),
    )(flp, hnow, t2p.reshape(2, F, NU, 1, UB), fa0, sflp, g, lfla1)


def kernel(flp, hnow, w1, w2, la1, la2, r1, r2):
    p1, p2 = _sc_edge_sums(r1, w1, r2, w2)
    inv1, inv2, cs2 = _tc_prep_sums(la1, la2, p1, p2)

    lfw1 = _sc_lfw1(inv1.reshape(U), r1, w1)
    lfw2, fa0p = _sc_lfw2_fa0(inv2.reshape(U), r2, w2, r1, lfw1)

    lfla1, lfla2, flpT = _tc_normalize(la1, la2, inv1, cs2, flp)
    packed = _tc_pack(r1, r2)

    fa0 = None
    for dep in range(DEP):
        if dep == 0:
            g, sflp, fa0 = _tc_reduce(flp, lfla2, fa0p)
        else:
            g, sflp = _tc_reduce(flp, lfla2)
        tp = _sc_spmm(flpT, packed, lfw2, gather_hi=False)
        t2p = _sc_spmm(tp, packed, lfw1, gather_hi=True, paired_src=True)
        last = dep == DEP - 1
        if last:
            (flp,) = _tc_update(flp, hnow, t2p, fa0, sflp, g, lfla1, last=True)
        else:
            flp, flpT = _tc_update(flp, hnow, t2p, fa0, sflp, g, lfla1,
                                   last=False)

    return (lfw1, lfw2, lfla1, lfla2, flp)
